# baseline (speedup 1.0000x reference)
import functools

import jax
import jax.numpy as jnp
import numpy as np
from jax.experimental import pallas as pl
from jax.experimental.pallas import tpu as pltpu

D = 32
H = 4
BN = 4096


def _stats_body(n_valid, lig_ref, rec_ref, m_out, s_out, m_sc, s_sc):
    i = pl.program_id(0)
    nb = pl.num_programs(0)

    @pl.when(i == 0)
    def _init():
        m_sc[...] = jnp.full_like(m_sc, -jnp.inf)
        s_sc[...] = jnp.zeros_like(s_sc)

    d = jax.lax.dot_general(lig_ref[...], rec_ref[...],
                            (((1,), (1,)), ((), ())),
                            preferred_element_type=jnp.float32)
    col = i * BN + jax.lax.broadcasted_iota(jnp.int32, d.shape, 1)
    d = jnp.where(col < n_valid, d, -jnp.inf)
    bm = jnp.max(d, axis=1, keepdims=True)
    m_old = m_sc[...]
    m_new = jnp.maximum(m_old, bm)
    s_sc[...] = s_sc[...] * jnp.exp(m_old - m_new) + jnp.sum(
        jnp.exp(d - m_new), axis=1, keepdims=True)
    m_sc[...] = m_new

    @pl.when(i == nb - 1)
    def _fin():
        m_out[...] = m_sc[...]
        s_out[...] = s_sc[...]


def _readout_body(n_valid, lig_ref, m_ref, s_ref, rec_ref, xyz_ref, a_out, y_out, y_sc):
    i = pl.program_id(0)
    nb = pl.num_programs(0)

    @pl.when(i == 0)
    def _init():
        y_sc[...] = jnp.zeros_like(y_sc)

    d = jax.lax.dot_general(lig_ref[...], rec_ref[...],
                            (((1,), (1,)), ((), ())),
                            preferred_element_type=jnp.float32)
    col = i * BN + jax.lax.broadcasted_iota(jnp.int32, d.shape, 1)
    a = jnp.exp(d - m_ref[...]) / s_ref[...]
    a = jnp.where(col < n_valid, a, 0.0)
    a_out[...] = a
    y_sc[...] += jax.lax.dot_general(a, xyz_ref[...],
                                     (((1,), (0,)), ((), ())),
                                     preferred_element_type=jnp.float32)

    @pl.when(i == nb - 1)
    def _fin():
        y_out[...] = y_sc[...]


def _readout(hs_rec, hs_lig_k, xyz_rec):
    n = hs_rec.shape[0]
    k = hs_lig_k.shape[0]
    n_pad = ((n + BN - 1) // BN) * BN
    if n_pad != n:
        hs_rec = jnp.pad(hs_rec, ((0, n_pad - n), (0, 0)))
        xyz_rec = jnp.pad(xyz_rec, ((0, n_pad - n), (0, 0)))
    nb = n_pad // BN
    m, s = pl.pallas_call(
        functools.partial(_stats_body, n),
        grid=(nb,),
        in_specs=[
            pl.BlockSpec((k, D), lambda i: (0, 0)),
            pl.BlockSpec((BN, D), lambda i: (i, 0)),
        ],
        out_specs=[
            pl.BlockSpec((k, 1), lambda i: (0, 0)),
            pl.BlockSpec((k, 1), lambda i: (0, 0)),
        ],
        out_shape=[
            jax.ShapeDtypeStruct((k, 1), jnp.float32),
            jax.ShapeDtypeStruct((k, 1), jnp.float32),
        ],
        scratch_shapes=[
            pltpu.VMEM((k, 1), jnp.float32),
            pltpu.VMEM((k, 1), jnp.float32),
        ],
    )(hs_lig_k, hs_rec)
    a, y = pl.pallas_call(
        functools.partial(_readout_body, n),
        grid=(nb,),
        in_specs=[
            pl.BlockSpec((k, D), lambda i: (0, 0)),
            pl.BlockSpec((k, 1), lambda i: (0, 0)),
            pl.BlockSpec((k, 1), lambda i: (0, 0)),
            pl.BlockSpec((BN, D), lambda i: (i, 0)),
            pl.BlockSpec((BN, 3), lambda i: (i, 0)),
        ],
        out_specs=[
            pl.BlockSpec((k, BN), lambda i: (0, i)),
            pl.BlockSpec((k, 3), lambda i: (0, 0)),
        ],
        out_shape=[
            jax.ShapeDtypeStruct((k, n_pad), jnp.float32),
            jax.ShapeDtypeStruct((k, 3), jnp.float32),
        ],
        scratch_shapes=[
            pltpu.VMEM((k, 3), jnp.float32),
        ],
    )(hs_lig_k, m, s, hs_rec, xyz_rec)
    return y, a[:, :n]


def _segment_softmax(logits, seg, n):
    m = jax.ops.segment_max(logits, seg, num_segments=n)
    m = jnp.where(jnp.isfinite(m), m, 0.0)
    e = jnp.exp(logits - m[seg])
    dnm = jax.ops.segment_sum(e, seg, num_segments=n)
    return e / (dnm[seg] + 1e-9)


def _attn_layer(h, src, dst, efeat, p, n):
    dh = D // H
    q = (h @ p['Wq'])[dst].reshape(-1, H, dh)
    k = (h[src] @ p['Wk'] + efeat @ p['Wek']).reshape(-1, H, dh)
    v = (h[src] @ p['Wv'] + efeat @ p['Wev']).reshape(-1, H, dh)
    logits = (q * k).sum(-1) / np.sqrt(dh)
    a = _segment_softmax(logits, dst, n)
    msg = (a[..., None] * v).reshape(-1, D)
    agg = jax.ops.segment_sum(msg, dst, num_segments=n)
    return jax.nn.relu(h @ p['Ws'] + agg @ p['Wo'])


def _encode(attr, eattr, eidx, p, n):
    src, dst = eidx[0], eidx[1]
    h = jax.nn.relu(attr @ p['Win'])
    for lp in p['layers']:
        h = _attn_layer(h, src, dst, eattr, lp, n)
    return h @ p['Wout']


def kernel(rec_attr, rec_edge_attr, rec_x, lig_attr, lig_edge_attr, labelidx, params, rec_edge_index, lig_edge_index):
    hs_rec = _encode(rec_attr, rec_edge_attr, rec_edge_index, params['rec'], rec_attr.shape[0])
    hs_lig = _encode(lig_attr, lig_edge_attr, lig_edge_index, params['lig'], lig_attr.shape[0])
    xyz_rec = rec_x.reshape(-1, 3)
    hs_lig = jax.nn.relu(hs_lig @ params['phi_W'] + params['phi_b'])
    hs_lig = labelidx @ hs_lig
    return _readout(hs_rec, hs_lig, xyz_rec)


# speedup vs baseline: 2.9263x; 2.9263x over previous
import functools

import jax
import jax.numpy as jnp
import numpy as np
from jax import lax
from jax.experimental import pallas as pl
from jax.experimental.pallas import tpu as pltpu
from jax.experimental.pallas import tpu_sc as plsc

D = 32
H = 4
NC = 2
NS = 16
NW = NC * NS
BE = 5000
BN = 4096


def _pick_chunk(per, align):
    for c in (1024, 800, 512, 400, 256, 200, 160, 128, 100, 80, 64, 32, 16):
        if c % align == 0 and per % c == 0:
            return c
    raise ValueError(f"no chunk for {per} align {align}")


def _mesh():
    return plsc.VectorSubcoreMesh(core_axis_name="c", subcore_axis_name="s",
                                  num_cores=NC, num_subcores=NS)


_SC_PARAMS = pltpu.CompilerParams(use_tc_tiling_on_sc=False)


def _sc_gather3(kn, vn, qs, src, dst):
    e = src.shape[0]
    qw = qs.shape[1]
    per_w = e // NW
    c_sz = _pick_chunk(per_w, 8)
    rounds = per_w // c_sz

    @functools.partial(
        pl.kernel,
        out_type=[jax.ShapeDtypeStruct((e, D), jnp.float32),
                  jax.ShapeDtypeStruct((e, D), jnp.float32),
                  jax.ShapeDtypeStruct((e, qw), jnp.float32)],
        mesh=_mesh(),
        compiler_params=_SC_PARAMS,
        scratch_types=[pltpu.VMEM((c_sz,), jnp.int32),
                       pltpu.VMEM((c_sz,), jnp.int32),
                       pltpu.VMEM((c_sz, D), jnp.float32),
                       pltpu.VMEM((c_sz, D), jnp.float32),
                       pltpu.VMEM((c_sz, qw), jnp.float32),
                       pltpu.SemaphoreType.DMA,
                       pltpu.SemaphoreType.DMA,
                       pltpu.SemaphoreType.DMA],
    )
    def k(kn_h, vn_h, qs_h, src_h, dst_h, ko, vo, qo, si, di, kb, vb, qb,
          s1, s2, s3):
        w = lax.axis_index("s") * NC + lax.axis_index("c")
        base0 = w * per_w

        def body(r, carry):
            base = base0 + r * c_sz
            pltpu.sync_copy(src_h.at[pl.ds(base, c_sz)], si)
            pltpu.sync_copy(dst_h.at[pl.ds(base, c_sz)], di)
            c1 = pltpu.async_copy(kn_h.at[si], kb, s1)
            c2 = pltpu.async_copy(vn_h.at[si], vb, s2)
            c3 = pltpu.async_copy(qs_h.at[di], qb, s3)
            c1.wait()
            c2.wait()
            c3.wait()
            pltpu.sync_copy(kb, ko.at[pl.ds(base, c_sz)])
            pltpu.sync_copy(vb, vo.at[pl.ds(base, c_sz)])
            pltpu.sync_copy(qb, qo.at[pl.ds(base, c_sz)])
            return carry

        lax.fori_loop(0, rounds, body, 0)

    return k(kn, vn, qs, src, dst)


def _sc_gather1(table, idx):
    e = idx.shape[0]
    w_cols = table.shape[1]
    per_w = e // NW
    c_sz = _pick_chunk(per_w, 8)
    rounds = per_w // c_sz

    @functools.partial(
        pl.kernel,
        out_type=jax.ShapeDtypeStruct((e, w_cols), jnp.float32),
        mesh=_mesh(),
        compiler_params=_SC_PARAMS,
        scratch_types=[pltpu.VMEM((c_sz,), jnp.int32),
                       pltpu.VMEM((c_sz, w_cols), jnp.float32),
                       pltpu.SemaphoreType.DMA],
    )
    def k(tab_h, idx_h, out, iv, rb, sem):
        w = lax.axis_index("s") * NC + lax.axis_index("c")
        base0 = w * per_w

        def body(r, carry):
            base = base0 + r * c_sz
            pltpu.sync_copy(idx_h.at[pl.ds(base, c_sz)], iv)
            pltpu.async_copy(tab_h.at[iv], rb, sem).wait()
            pltpu.sync_copy(rb, out.at[pl.ds(base, c_sz)])
            return carry

        lax.fori_loop(0, rounds, body, 0)

    return k(table, idx)


def _sc_scatter_denom(el16, dst, n, zeros_n16):
    e = dst.shape[0]
    per_w = e // NW
    c_sz = _pick_chunk(per_w, 8)
    rounds = per_w // c_sz

    @functools.partial(
        pl.kernel,
        out_type=jax.ShapeDtypeStruct((NC, n, 16), jnp.float32),
        mesh=_mesh(),
        compiler_params=_SC_PARAMS,
        scratch_types=[pltpu.VMEM((c_sz,), jnp.int32),
                       pltpu.VMEM((c_sz, 16), jnp.float32),
                       pltpu.VMEM_SHARED((n, 16), jnp.float32)],
    )
    def k(el_h, dst_h, z_h, out, iv, rb, acc):
        c = lax.axis_index("c")
        s = lax.axis_index("s")
        w = s * NC + c

        @pl.when(s == 0)
        def _zero():
            pltpu.sync_copy(z_h, acc)

        plsc.subcore_barrier()

        def body(r, carry):
            base = w * per_w + r * c_sz
            pltpu.sync_copy(dst_h.at[pl.ds(base, c_sz)], iv)
            pltpu.sync_copy(el_h.at[pl.ds(base, c_sz)], rb)
            pltpu.sync_copy(rb, acc.at[iv], add=True)
            return carry

        lax.fori_loop(0, rounds, body, 0)
        plsc.subcore_barrier()

        @pl.when(s == 0)
        def _dump():
            pltpu.sync_copy(acc, out.at[c])

    return k(el16, dst, zeros_n16)


def _sc_scatter_msg(msg, dst, n, zeros_acc):
    e = dst.shape[0]
    n2 = n // NC
    npad = n2 + 8
    per_t = e // NS
    c_sz = _pick_chunk(per_t, 16)
    rounds = per_t // c_sz

    @functools.partial(
        pl.kernel,
        out_type=jax.ShapeDtypeStruct((n, D), jnp.float32),
        mesh=_mesh(),
        compiler_params=_SC_PARAMS,
        scratch_types=[pltpu.VMEM((c_sz,), jnp.int32),
                       pltpu.VMEM((c_sz, D), jnp.float32),
                       pltpu.VMEM_SHARED((npad, D), jnp.float32)],
    )
    def k(msg_h, dst_h, z_h, out, iv, rb, acc):
        c = lax.axis_index("c")
        s = lax.axis_index("s")
        lo = c * n2

        @pl.when(s == 0)
        def _zero():
            pltpu.sync_copy(z_h, acc)

        plsc.subcore_barrier()

        def body(r, carry):
            base = s * per_t + r * c_sz
            pltpu.sync_copy(dst_h.at[pl.ds(base, c_sz)], iv)
            for g in range(c_sz // 16):
                v = iv[pl.ds(g * 16, 16)]
                loc = v - lo
                ok = (loc >= 0) & (loc < n2)
                iv[pl.ds(g * 16, 16)] = jnp.where(ok, loc, n2)
            pltpu.sync_copy(msg_h.at[pl.ds(base, c_sz)], rb)
            pltpu.sync_copy(rb, acc.at[iv], add=True)
            return carry

        lax.fori_loop(0, rounds, body, 0)
        plsc.subcore_barrier()

        @pl.when(s == 0)
        def _dump():
            pltpu.sync_copy(acc.at[pl.ds(0, n2)], out.at[pl.ds(lo, n2)])

    return k(msg, dst, zeros_acc)


def _grid1(n_rows, blk):
    assert n_rows % blk == 0, (n_rows, blk)
    return n_rows // blk


def _row_spec(blk, w):
    return pl.BlockSpec((blk, w), lambda i: (i, 0))


def _full_spec(a, b):
    return pl.BlockSpec((a, b), lambda i: (0, 0))


def _lin_relu_body(a_ref, w_ref, b_ref, o_ref):
    o_ref[...] = jax.nn.relu(
        jax.lax.dot_general(a_ref[...], w_ref[...], (((1,), (0,)), ((), ())),
                            preferred_element_type=jnp.float32) + b_ref[...])


def _lin_relu(a, w, b):
    n, din = a.shape
    g = _grid1(n, BE)
    return pl.pallas_call(
        _lin_relu_body,
        grid=(g,),
        in_specs=[_row_spec(BE, din), _full_spec(din, D), _full_spec(1, D)],
        out_specs=_row_spec(BE, D),
        out_shape=jax.ShapeDtypeStruct((n, D), jnp.float32),
    )(a, w, b.reshape(1, D))


def _mm_body(a_ref, w_ref, o_ref):
    o_ref[...] = jax.lax.dot_general(
        a_ref[...], w_ref[...], (((1,), (0,)), ((), ())),
        preferred_element_type=jnp.float32)


def _mm(a, w):
    n = a.shape[0]
    g = _grid1(n, BE)
    return pl.pallas_call(
        _mm_body,
        grid=(g,),
        in_specs=[_row_spec(BE, D), _full_spec(D, w.shape[1])],
        out_specs=_row_spec(BE, w.shape[1]),
        out_shape=jax.ShapeDtypeStruct((n, w.shape[1]), jnp.float32),
    )(a, w)


def _proj_body(h_ref, wq, wk, wv, wsp, qs_o, k_o, v_o):
    hh = h_ref[...]
    dn = (((1,), (0,)), ((), ()))
    q = jax.lax.dot_general(hh, wq[...], dn, preferred_element_type=jnp.float32)
    k_o[...] = jax.lax.dot_general(hh, wk[...], dn,
                                   preferred_element_type=jnp.float32)
    v_o[...] = jax.lax.dot_general(hh, wv[...], dn,
                                   preferred_element_type=jnp.float32)
    s = jax.lax.dot_general(q, wsp[...], dn,
                            preferred_element_type=jnp.float32,
                            precision=lax.Precision.HIGHEST)
    qs_o[...] = jnp.concatenate([q, s], axis=1)


def _proj(h, wq, wk, wv, wsp, qw):
    n = h.shape[0]
    g = _grid1(n, BE)
    sw = qw - D
    return pl.pallas_call(
        _proj_body,
        grid=(g,),
        in_specs=[_row_spec(BE, D), _full_spec(D, D), _full_spec(D, D),
                  _full_spec(D, D), _full_spec(D, sw)],
        out_specs=[_row_spec(BE, qw), _row_spec(BE, D), _row_spec(BE, D)],
        out_shape=[jax.ShapeDtypeStruct((n, qw), jnp.float32),
                   jax.ShapeDtypeStruct((n, D), jnp.float32),
                   jax.ShapeDtypeStruct((n, D), jnp.float32)],
    )(h, wq, wk, wv, wsp)


def _edge_logits_body(d_e, ks_ref, vs_ref, qs_ref, ea_ref, hm_ref, wev_ref,
                      el_o, ve_o):
    q = qs_ref[:, :D]
    p = q * ks_ref[...]
    l = jax.lax.dot_general(p, hm_ref[...], (((1,), (0,)), ((), ())),
                            preferred_element_type=jnp.float32,
                            precision=lax.Precision.HIGHEST)
    ea = ea_ref[...]
    if d_e > 1:
        ea = ea.astype(jnp.bfloat16).astype(jnp.float32)
    for j in range(d_e):
        l = l + ea[:, j:j + 1] * qs_ref[:, D + H * j:D + H * j + H]
    l = l * np.float32(1.0 / np.sqrt(D // H))
    el = jnp.exp(l)
    el_o[...] = jnp.concatenate(
        [el, jnp.zeros((el.shape[0], 12), jnp.float32)], axis=1)
    wev = wev_ref[...]
    if d_e > 1:
        wev = wev.astype(jnp.bfloat16).astype(jnp.float32)
    ve = vs_ref[...]
    for j in range(d_e):
        ve = ve + ea[:, j:j + 1] * wev[j:j + 1, :]
    ve_o[...] = ve


def _edge_logits(ksrc, vsrc, qsdst, eattr, hmask, wev):
    e, d_e = eattr.shape
    qw = qsdst.shape[1]
    g = _grid1(e, BE)
    return pl.pallas_call(
        functools.partial(_edge_logits_body, d_e),
        grid=(g,),
        in_specs=[_row_spec(BE, D), _row_spec(BE, D), _row_spec(BE, qw),
                  _row_spec(BE, d_e), _full_spec(D, H), _full_spec(d_e, D)],
        out_specs=[_row_spec(BE, 16), _row_spec(BE, D)],
        out_shape=[jax.ShapeDtypeStruct((e, 16), jnp.float32),
                   jax.ShapeDtypeStruct((e, D), jnp.float32)],
    )(ksrc, vsrc, qsdst, eattr, hmask, wev)


def _add2_body(a_ref, b_ref, o_ref):
    o_ref[...] = a_ref[...] + b_ref[...]


def _add2(a, b):
    n = a.shape[0]
    g = _grid1(n, BE)
    return pl.pallas_call(
        _add2_body,
        grid=(g,),
        in_specs=[_row_spec(BE, a.shape[1]), _row_spec(BE, a.shape[1])],
        out_specs=_row_spec(BE, a.shape[1]),
        out_shape=jax.ShapeDtypeStruct(a.shape, jnp.float32),
    )(a, b)


def _msg_body(el_ref, dd_ref, ve_ref, r_ref, o_ref):
    w = el_ref[:, :H] / (dd_ref[:, :H] + 1e-9)
    o_ref[...] = jax.lax.dot_general(
        w, r_ref[...], (((1,), (0,)), ((), ())),
        preferred_element_type=jnp.float32,
        precision=lax.Precision.HIGHEST) * ve_ref[...]


def _msg(el16, ddst, ve, rexp):
    e = ve.shape[0]
    g = _grid1(e, BE)
    return pl.pallas_call(
        _msg_body,
        grid=(g,),
        in_specs=[_row_spec(BE, 16), _row_spec(BE, 16), _row_spec(BE, D),
                  _full_spec(H, D)],
        out_specs=_row_spec(BE, D),
        out_shape=jax.ShapeDtypeStruct((e, D), jnp.float32),
    )(el16, ddst, ve, rexp)


def _epi_body(h_ref, agg_ref, ws, wo, o_ref):
    dn = (((1,), (0,)), ((), ()))
    o_ref[...] = jax.nn.relu(
        jax.lax.dot_general(h_ref[...], ws[...], dn,
                            preferred_element_type=jnp.float32)
        + jax.lax.dot_general(agg_ref[...], wo[...], dn,
                              preferred_element_type=jnp.float32))


def _epi(h, agg, ws, wo):
    n = h.shape[0]
    g = _grid1(n, BE)
    return pl.pallas_call(
        _epi_body,
        grid=(g,),
        in_specs=[_row_spec(BE, D), _row_spec(BE, D), _full_spec(D, D),
                  _full_spec(D, D)],
        out_specs=_row_spec(BE, D),
        out_shape=jax.ShapeDtypeStruct((n, D), jnp.float32),
    )(h, agg, ws, wo)


def _lab_body(li_ref, hl_ref, pw, pb, o_ref):
    dn = (((1,), (0,)), ((), ()))
    hl = jax.nn.relu(
        jax.lax.dot_general(hl_ref[...], pw[...], dn,
                            preferred_element_type=jnp.float32) + pb[...])
    o_ref[...] = jax.lax.dot_general(li_ref[...], hl, dn,
                                     preferred_element_type=jnp.float32)


def _label_contract(labelidx, hs_lig, phi_w, phi_b):
    k, n = labelidx.shape
    return pl.pallas_call(
        _lab_body,
        grid=(1,),
        in_specs=[_full_spec(k, n), _full_spec(n, D),
                  _full_spec(D, D), _full_spec(1, D)],
        out_specs=pl.BlockSpec((k, D), lambda i: (0, 0)),
        out_shape=jax.ShapeDtypeStruct((k, D), jnp.float32),
    )(labelidx, hs_lig, phi_w, phi_b.reshape(1, D))


def _stats_body(n_valid, lig_ref, rec_ref, m_out, s_out, m_sc, s_sc):
    i = pl.program_id(0)
    nb = pl.num_programs(0)

    @pl.when(i == 0)
    def _init():
        m_sc[...] = jnp.full_like(m_sc, -jnp.inf)
        s_sc[...] = jnp.zeros_like(s_sc)

    d = jax.lax.dot_general(lig_ref[...], rec_ref[...],
                            (((1,), (1,)), ((), ())),
                            preferred_element_type=jnp.float32)
    col = i * BN + jax.lax.broadcasted_iota(jnp.int32, d.shape, 1)
    d = jnp.where(col < n_valid, d, -jnp.inf)
    bm = jnp.max(d, axis=1, keepdims=True)
    m_old = m_sc[...]
    m_new = jnp.maximum(m_old, bm)
    s_sc[...] = s_sc[...] * jnp.exp(m_old - m_new) + jnp.sum(
        jnp.exp(d - m_new), axis=1, keepdims=True)
    m_sc[...] = m_new

    @pl.when(i == nb - 1)
    def _fin():
        m_out[...] = m_sc[...]
        s_out[...] = s_sc[...]


def _readout_body(n_valid, lig_ref, m_ref, s_ref, rec_ref, xyz_ref, a_out,
                  y_out, y_sc):
    i = pl.program_id(0)
    nb = pl.num_programs(0)

    @pl.when(i == 0)
    def _init():
        y_sc[...] = jnp.zeros_like(y_sc)

    d = jax.lax.dot_general(lig_ref[...], rec_ref[...],
                            (((1,), (1,)), ((), ())),
                            preferred_element_type=jnp.float32)
    col = i * BN + jax.lax.broadcasted_iota(jnp.int32, d.shape, 1)
    a = jnp.exp(d - m_ref[...]) / s_ref[...]
    a = jnp.where(col < n_valid, a, 0.0)
    a_out[...] = a
    y_sc[...] += jax.lax.dot_general(a, xyz_ref[...],
                                     (((1,), (0,)), ((), ())),
                                     preferred_element_type=jnp.float32)

    @pl.when(i == nb - 1)
    def _fin():
        y_out[...] = y_sc[...]


def _readout(hs_rec, hs_lig_k, xyz_rec):
    n = hs_rec.shape[0]
    k = hs_lig_k.shape[0]
    n_pad = ((n + BN - 1) // BN) * BN
    if n_pad != n:
        hs_rec = jnp.pad(hs_rec, ((0, n_pad - n), (0, 0)))
        xyz_rec = jnp.pad(xyz_rec, ((0, n_pad - n), (0, 0)))
    nb = n_pad // BN
    m, s = pl.pallas_call(
        functools.partial(_stats_body, n),
        grid=(nb,),
        in_specs=[
            pl.BlockSpec((k, D), lambda i: (0, 0)),
            pl.BlockSpec((BN, D), lambda i: (i, 0)),
        ],
        out_specs=[
            pl.BlockSpec((k, 1), lambda i: (0, 0)),
            pl.BlockSpec((k, 1), lambda i: (0, 0)),
        ],
        out_shape=[
            jax.ShapeDtypeStruct((k, 1), jnp.float32),
            jax.ShapeDtypeStruct((k, 1), jnp.float32),
        ],
        scratch_shapes=[
            pltpu.VMEM((k, 1), jnp.float32),
            pltpu.VMEM((k, 1), jnp.float32),
        ],
    )(hs_lig_k, hs_rec)
    a, y = pl.pallas_call(
        functools.partial(_readout_body, n),
        grid=(nb,),
        in_specs=[
            pl.BlockSpec((k, D), lambda i: (0, 0)),
            pl.BlockSpec((k, 1), lambda i: (0, 0)),
            pl.BlockSpec((k, 1), lambda i: (0, 0)),
            pl.BlockSpec((BN, D), lambda i: (i, 0)),
            pl.BlockSpec((BN, 3), lambda i: (i, 0)),
        ],
        out_specs=[
            pl.BlockSpec((k, BN), lambda i: (0, i)),
            pl.BlockSpec((k, 3), lambda i: (0, 0)),
        ],
        out_shape=[
            jax.ShapeDtypeStruct((k, n_pad), jnp.float32),
            jax.ShapeDtypeStruct((k, 3), jnp.float32),
        ],
        scratch_shapes=[
            pltpu.VMEM((k, 3), jnp.float32),
        ],
    )(hs_lig_k, m, s, hs_rec, xyz_rec)
    return y, a[:, :n]


def _head_mask():
    return (jnp.arange(D)[:, None] // (D // H)
            == jnp.arange(H)[None, :]).astype(jnp.float32)


def _attn_layer_sc(h, src, dst, eattr, p, n, zeros_n16, zeros_acc):
    d_e = eattr.shape[1]
    hmask = _head_mask()
    sw_used = H * d_e
    qw = D + ((sw_used + 15) // 16) * 16
    wek = p['Wek']
    if d_e > 1:
        wek = wek.astype(jnp.bfloat16).astype(jnp.float32)
    ws = jnp.concatenate(
        [wek[j][:, None] * hmask for j in range(d_e)], axis=1)
    ws = jnp.pad(ws, ((0, 0), (0, (qw - D) - sw_used)))

    qs, kn, vn = _proj(h, p['Wq'], p['Wk'], p['Wv'], ws, qw)
    ksrc, vsrc, qsdst = _sc_gather3(kn, vn, qs, src, dst)
    el16, ve = _edge_logits(ksrc, vsrc, qsdst, eattr, hmask, p['Wev'])
    dpart = _sc_scatter_denom(el16, dst, n, zeros_n16)
    denom = _add2(dpart[0], dpart[1])
    ddst = _sc_gather1(denom, dst)
    msg = _msg(el16, ddst, ve, hmask.T)
    agg = _sc_scatter_msg(msg, dst, n, zeros_acc)
    return _epi(h, agg, p['Ws'], p['Wo'])


def _encode_sc(attr, eattr, eidx, p, n):
    src, dst = eidx[0], eidx[1]
    zeros_n16 = jnp.zeros((n, 16), jnp.float32)
    zeros_acc = jnp.zeros((n // NC + 8, D), jnp.float32)
    zb = jnp.zeros((1, D), jnp.float32)
    h = _lin_relu(attr, p['Win'], zb)
    for lp in p['layers']:
        h = _attn_layer_sc(h, src, dst, eattr, lp, n, zeros_n16, zeros_acc)
    return _mm(h, p['Wout'])


def kernel(rec_attr, rec_edge_attr, rec_x, lig_attr, lig_edge_attr, labelidx,
           params, rec_edge_index, lig_edge_index):
    hs_rec = _encode_sc(rec_attr, rec_edge_attr, rec_edge_index,
                        params['rec'], rec_attr.shape[0])
    hs_lig = _encode_sc(lig_attr, lig_edge_attr, lig_edge_index,
                        params['lig'], lig_attr.shape[0])
    xyz_rec = rec_x.reshape(-1, 3)
    hs_lig_k = _label_contract(labelidx, hs_lig, params['phi_W'],
                               params['phi_b'])
    return _readout(hs_rec, hs_lig_k, xyz_rec)


# speedup vs baseline: 2.9982x; 1.0246x over previous
import functools

import jax
import jax.numpy as jnp
import numpy as np
from jax import lax
from jax.experimental import pallas as pl
from jax.experimental.pallas import tpu as pltpu
from jax.experimental.pallas import tpu_sc as plsc

D = 32
H = 4
NC = 2
NS = 16
NW = NC * NS
BE = 5000
BN = 4096


def _pick_chunk(per, align):
    for c in (5000, 4000, 2000, 1600, 1000, 800, 512, 400, 256, 200, 160, 128, 100, 80, 64, 32, 16):
        if c % align == 0 and per % c == 0:
            return c
    raise ValueError(f"no chunk for {per} align {align}")


def _mesh():
    return plsc.VectorSubcoreMesh(core_axis_name="c", subcore_axis_name="s",
                                  num_cores=NC, num_subcores=NS)


_SC_PARAMS = pltpu.CompilerParams(use_tc_tiling_on_sc=False)


def _sc_gather_kq(kn, qs, src, dst):
    e = src.shape[0]
    qw = qs.shape[1]
    per_w = e // NW
    c_sz = 1000
    assert per_w % c_sz == 0
    rounds = per_w // c_sz

    @functools.partial(
        pl.kernel,
        out_type=[jax.ShapeDtypeStruct((e, D), jnp.float32),
                  jax.ShapeDtypeStruct((e, qw), jnp.float32)],
        mesh=_mesh(),
        compiler_params=_SC_PARAMS,
        scratch_types=[pltpu.VMEM((c_sz,), jnp.int32),
                       pltpu.VMEM((c_sz,), jnp.int32),
                       pltpu.VMEM((c_sz, D), jnp.float32),
                       pltpu.VMEM((c_sz, qw), jnp.float32),
                       pltpu.SemaphoreType.DMA,
                       pltpu.SemaphoreType.DMA],
    )
    def k(kn_h, qs_h, src_h, dst_h, ko, qo, si, di, kb, qb, s1, s2):
        w = lax.axis_index("s") * NC + lax.axis_index("c")
        base0 = w * per_w

        def body(r, carry):
            base = base0 + r * c_sz
            pltpu.sync_copy(src_h.at[pl.ds(base, c_sz)], si)
            pltpu.sync_copy(dst_h.at[pl.ds(base, c_sz)], di)
            c1 = pltpu.async_copy(kn_h.at[si], kb, s1)
            c2 = pltpu.async_copy(qs_h.at[di], qb, s2)
            c1.wait()
            c2.wait()
            pltpu.sync_copy(kb, ko.at[pl.ds(base, c_sz)])
            pltpu.sync_copy(qb, qo.at[pl.ds(base, c_sz)])
            return carry

        lax.fori_loop(0, rounds, body, 0)

    return k(kn, qs, src, dst)


def _sc_gather_dv(den, vn, src, dst):
    e = src.shape[0]
    per_w = e // NW
    c_sz = 1000
    assert per_w % c_sz == 0
    rounds = per_w // c_sz

    @functools.partial(
        pl.kernel,
        out_type=[jax.ShapeDtypeStruct((e, 16), jnp.float32),
                  jax.ShapeDtypeStruct((e, D), jnp.float32)],
        mesh=_mesh(),
        compiler_params=_SC_PARAMS,
        scratch_types=[pltpu.VMEM((c_sz,), jnp.int32),
                       pltpu.VMEM((c_sz,), jnp.int32),
                       pltpu.VMEM((c_sz, 16), jnp.float32),
                       pltpu.VMEM((c_sz, D), jnp.float32),
                       pltpu.SemaphoreType.DMA,
                       pltpu.SemaphoreType.DMA],
    )
    def k(den_h, vn_h, src_h, dst_h, do, vo, si, di, db, vb, s1, s2):
        w = lax.axis_index("s") * NC + lax.axis_index("c")
        base0 = w * per_w

        def body(r, carry):
            base = base0 + r * c_sz
            pltpu.sync_copy(src_h.at[pl.ds(base, c_sz)], si)
            pltpu.sync_copy(dst_h.at[pl.ds(base, c_sz)], di)
            c1 = pltpu.async_copy(den_h.at[di], db, s1)
            c2 = pltpu.async_copy(vn_h.at[si], vb, s2)
            c1.wait()
            c2.wait()
            pltpu.sync_copy(db, do.at[pl.ds(base, c_sz)])
            pltpu.sync_copy(vb, vo.at[pl.ds(base, c_sz)])
            return carry

        lax.fori_loop(0, rounds, body, 0)

    return k(den, vn, src, dst)


def _sc_scatter_denom(el16, dst, n, zeros_n16):
    e = dst.shape[0]
    per_w = e // NW
    c_sz = 1000
    assert per_w % c_sz == 0
    rounds = per_w // c_sz

    @functools.partial(
        pl.kernel,
        out_type=jax.ShapeDtypeStruct((NC, n, 16), jnp.float32),
        mesh=_mesh(),
        compiler_params=_SC_PARAMS,
        scratch_types=[pltpu.VMEM((c_sz,), jnp.int32),
                       pltpu.VMEM((c_sz, 16), jnp.float32),
                       pltpu.VMEM_SHARED((n, 16), jnp.float32)],
    )
    def k(el_h, dst_h, z_h, out, iv, rb, acc):
        c = lax.axis_index("c")
        s = lax.axis_index("s")
        w = s * NC + c

        @pl.when(s == 0)
        def _zero():
            pltpu.sync_copy(z_h, acc)

        plsc.subcore_barrier()

        def body(r, carry):
            base = w * per_w + r * c_sz
            pltpu.sync_copy(dst_h.at[pl.ds(base, c_sz)], iv)
            pltpu.sync_copy(el_h.at[pl.ds(base, c_sz)], rb)
            pltpu.sync_copy(rb, acc.at[iv], add=True)
            return carry

        lax.fori_loop(0, rounds, body, 0)
        plsc.subcore_barrier()

        @pl.when(s == 0)
        def _dump():
            pltpu.sync_copy(acc, out.at[c])

    return k(el16, dst, zeros_n16)


def _sc_scatter_msg(msg, dst, n, zeros_acc):
    e = dst.shape[0]
    n2 = n // NC
    npad = n2 + 8
    per_t = e // NS
    c_sz = 800 if per_t % 800 == 0 else 400
    rounds = per_t // c_sz

    @functools.partial(
        pl.kernel,
        out_type=jax.ShapeDtypeStruct((n, D), jnp.float32),
        mesh=_mesh(),
        compiler_params=_SC_PARAMS,
        scratch_types=[pltpu.VMEM((c_sz,), jnp.int32),
                       pltpu.VMEM((c_sz, D), jnp.float32),
                       pltpu.VMEM_SHARED((npad, D), jnp.float32)],
    )
    def k(msg_h, dst_h, z_h, out, iv, rb, acc):
        c = lax.axis_index("c")
        s = lax.axis_index("s")
        lo = c * n2

        @pl.when(s == 0)
        def _zero():
            pltpu.sync_copy(z_h, acc)

        plsc.subcore_barrier()

        def body(r, carry):
            base = s * per_t + r * c_sz
            pltpu.sync_copy(dst_h.at[pl.ds(base, c_sz)], iv)
            for g in range(c_sz // 16):
                v = iv[pl.ds(g * 16, 16)]
                loc = v - lo
                ok = (loc >= 0) & (loc < n2)
                iv[pl.ds(g * 16, 16)] = jnp.where(ok, loc, n2)
            pltpu.sync_copy(msg_h.at[pl.ds(base, c_sz)], rb)
            pltpu.sync_copy(rb, acc.at[iv], add=True)
            return carry

        lax.fori_loop(0, rounds, body, 0)
        plsc.subcore_barrier()

        @pl.when(s == 0)
        def _dump():
            pltpu.sync_copy(acc.at[pl.ds(0, n2)], out.at[pl.ds(lo, n2)])

    return k(msg, dst, zeros_acc)


def _grid1(n_rows, blk):
    assert n_rows % blk == 0, (n_rows, blk)
    return n_rows // blk


def _row_spec(blk, w):
    return pl.BlockSpec((blk, w), lambda i: (i, 0))


def _full_spec(a, b):
    return pl.BlockSpec((a, b), lambda i: (0, 0))


def _lin_relu_body(a_ref, w_ref, b_ref, o_ref):
    o_ref[...] = jax.nn.relu(
        jax.lax.dot_general(a_ref[...], w_ref[...], (((1,), (0,)), ((), ())),
                            preferred_element_type=jnp.float32) + b_ref[...])


def _lin_relu(a, w, b):
    n, din = a.shape
    g = _grid1(n, BE)
    return pl.pallas_call(
        _lin_relu_body,
        grid=(g,),
        in_specs=[_row_spec(BE, din), _full_spec(din, D), _full_spec(1, D)],
        out_specs=_row_spec(BE, D),
        out_shape=jax.ShapeDtypeStruct((n, D), jnp.float32),
    )(a, w, b.reshape(1, D))


def _mm_body(a_ref, w_ref, o_ref):
    o_ref[...] = jax.lax.dot_general(
        a_ref[...], w_ref[...], (((1,), (0,)), ((), ())),
        preferred_element_type=jnp.float32)


def _mm(a, w):
    n = a.shape[0]
    g = _grid1(n, BE)
    return pl.pallas_call(
        _mm_body,
        grid=(g,),
        in_specs=[_row_spec(BE, D), _full_spec(D, w.shape[1])],
        out_specs=_row_spec(BE, w.shape[1]),
        out_shape=jax.ShapeDtypeStruct((n, w.shape[1]), jnp.float32),
    )(a, w)


def _proj_body(h_ref, wq, wk, wv, wsp, qs_o, k_o, v_o):
    hh = h_ref[...]
    dn = (((1,), (0,)), ((), ()))
    q = jax.lax.dot_general(hh, wq[...], dn, preferred_element_type=jnp.float32)
    k_o[...] = jax.lax.dot_general(hh, wk[...], dn,
                                   preferred_element_type=jnp.float32)
    v_o[...] = jax.lax.dot_general(hh, wv[...], dn,
                                   preferred_element_type=jnp.float32)
    s = jax.lax.dot_general(q, wsp[...], dn,
                            preferred_element_type=jnp.float32,
                            precision=lax.Precision.HIGHEST)
    qs_o[...] = jnp.concatenate([q, s], axis=1)


def _proj(h, wq, wk, wv, wsp, qw):
    n = h.shape[0]
    g = _grid1(n, BE)
    sw = qw - D
    return pl.pallas_call(
        _proj_body,
        grid=(g,),
        in_specs=[_row_spec(BE, D), _full_spec(D, D), _full_spec(D, D),
                  _full_spec(D, D), _full_spec(D, sw)],
        out_specs=[_row_spec(BE, qw), _row_spec(BE, D), _row_spec(BE, D)],
        out_shape=[jax.ShapeDtypeStruct((n, qw), jnp.float32),
                   jax.ShapeDtypeStruct((n, D), jnp.float32),
                   jax.ShapeDtypeStruct((n, D), jnp.float32)],
    )(h, wq, wk, wv, wsp)


def _edge_logits_body(d_e, ks_ref, qs_ref, ea_ref, hm_ref, el_o):
    q = qs_ref[:, :D]
    p = q * ks_ref[...]
    l = jax.lax.dot_general(p, hm_ref[...], (((1,), (0,)), ((), ())),
                            preferred_element_type=jnp.float32,
                            precision=lax.Precision.HIGHEST)
    ea = ea_ref[...]
    if d_e > 1:
        ea = ea.astype(jnp.bfloat16).astype(jnp.float32)
    for j in range(d_e):
        l = l + ea[:, j:j + 1] * qs_ref[:, D + H * j:D + H * j + H]
    l = l * np.float32(1.0 / np.sqrt(D // H))
    el = jnp.exp(l)
    el_o[...] = jnp.concatenate(
        [el, jnp.zeros((el.shape[0], 12), jnp.float32)], axis=1)


def _edge_logits(ksrc, qsdst, eattr, hmask):
    e, d_e = eattr.shape
    qw = qsdst.shape[1]
    g = _grid1(e, BE)
    return pl.pallas_call(
        functools.partial(_edge_logits_body, d_e),
        grid=(g,),
        in_specs=[_row_spec(BE, D), _row_spec(BE, qw),
                  _row_spec(BE, d_e), _full_spec(D, H)],
        out_specs=_row_spec(BE, 16),
        out_shape=jax.ShapeDtypeStruct((e, 16), jnp.float32),
    )(ksrc, qsdst, eattr, hmask)


def _add2_body(a_ref, b_ref, o_ref):
    o_ref[...] = a_ref[...] + b_ref[...]


def _add2(a, b):
    n = a.shape[0]
    g = _grid1(n, BE)
    return pl.pallas_call(
        _add2_body,
        grid=(g,),
        in_specs=[_row_spec(BE, a.shape[1]), _row_spec(BE, a.shape[1])],
        out_specs=_row_spec(BE, a.shape[1]),
        out_shape=jax.ShapeDtypeStruct(a.shape, jnp.float32),
    )(a, b)


def _msg_body(d_e, el_ref, dd_ref, vs_ref, ea_ref, wev_ref, r_ref, o_ref):
    w = el_ref[:, :H] / (dd_ref[:, :H] + 1e-9)
    ea = ea_ref[...]
    wev = wev_ref[...]
    if d_e > 1:
        ea = ea.astype(jnp.bfloat16).astype(jnp.float32)
        wev = wev.astype(jnp.bfloat16).astype(jnp.float32)
    ve = vs_ref[...]
    for j in range(d_e):
        ve = ve + ea[:, j:j + 1] * wev[j:j + 1, :]
    o_ref[...] = jax.lax.dot_general(
        w, r_ref[...], (((1,), (0,)), ((), ())),
        preferred_element_type=jnp.float32,
        precision=lax.Precision.HIGHEST) * ve


def _msg(el16, ddst, vsrc, eattr, wev, rexp):
    e, d_e = eattr.shape
    g = _grid1(e, BE)
    return pl.pallas_call(
        functools.partial(_msg_body, d_e),
        grid=(g,),
        in_specs=[_row_spec(BE, 16), _row_spec(BE, 16), _row_spec(BE, D),
                  _row_spec(BE, d_e), _full_spec(d_e, D), _full_spec(H, D)],
        out_specs=_row_spec(BE, D),
        out_shape=jax.ShapeDtypeStruct((e, D), jnp.float32),
    )(el16, ddst, vsrc, eattr, wev, rexp)


def _epi_body(h_ref, agg_ref, ws, wo, o_ref):
    dn = (((1,), (0,)), ((), ()))
    o_ref[...] = jax.nn.relu(
        jax.lax.dot_general(h_ref[...], ws[...], dn,
                            preferred_element_type=jnp.float32)
        + jax.lax.dot_general(agg_ref[...], wo[...], dn,
                              preferred_element_type=jnp.float32))


def _epi(h, agg, ws, wo):
    n = h.shape[0]
    g = _grid1(n, BE)
    return pl.pallas_call(
        _epi_body,
        grid=(g,),
        in_specs=[_row_spec(BE, D), _row_spec(BE, D), _full_spec(D, D),
                  _full_spec(D, D)],
        out_specs=_row_spec(BE, D),
        out_shape=jax.ShapeDtypeStruct((n, D), jnp.float32),
    )(h, agg, ws, wo)


def _lab_body(li_ref, hl_ref, pw, pb, o_ref):
    dn = (((1,), (0,)), ((), ()))
    hl = jax.nn.relu(
        jax.lax.dot_general(hl_ref[...], pw[...], dn,
                            preferred_element_type=jnp.float32) + pb[...])
    o_ref[...] = jax.lax.dot_general(li_ref[...], hl, dn,
                                     preferred_element_type=jnp.float32)


def _label_contract(labelidx, hs_lig, phi_w, phi_b):
    k, n = labelidx.shape
    return pl.pallas_call(
        _lab_body,
        grid=(1,),
        in_specs=[_full_spec(k, n), _full_spec(n, D),
                  _full_spec(D, D), _full_spec(1, D)],
        out_specs=pl.BlockSpec((k, D), lambda i: (0, 0)),
        out_shape=jax.ShapeDtypeStruct((k, D), jnp.float32),
    )(labelidx, hs_lig, phi_w, phi_b.reshape(1, D))


def _stats_body(n_valid, lig_ref, rec_ref, m_out, s_out, m_sc, s_sc):
    i = pl.program_id(0)
    nb = pl.num_programs(0)

    @pl.when(i == 0)
    def _init():
        m_sc[...] = jnp.full_like(m_sc, -jnp.inf)
        s_sc[...] = jnp.zeros_like(s_sc)

    d = jax.lax.dot_general(lig_ref[...], rec_ref[...],
                            (((1,), (1,)), ((), ())),
                            preferred_element_type=jnp.float32)
    col = i * BN + jax.lax.broadcasted_iota(jnp.int32, d.shape, 1)
    d = jnp.where(col < n_valid, d, -jnp.inf)
    bm = jnp.max(d, axis=1, keepdims=True)
    m_old = m_sc[...]
    m_new = jnp.maximum(m_old, bm)
    s_sc[...] = s_sc[...] * jnp.exp(m_old - m_new) + jnp.sum(
        jnp.exp(d - m_new), axis=1, keepdims=True)
    m_sc[...] = m_new

    @pl.when(i == nb - 1)
    def _fin():
        m_out[...] = m_sc[...]
        s_out[...] = s_sc[...]


def _readout_body(n_valid, lig_ref, m_ref, s_ref, rec_ref, xyz_ref, a_out,
                  y_out, y_sc):
    i = pl.program_id(0)
    nb = pl.num_programs(0)

    @pl.when(i == 0)
    def _init():
        y_sc[...] = jnp.zeros_like(y_sc)

    d = jax.lax.dot_general(lig_ref[...], rec_ref[...],
                            (((1,), (1,)), ((), ())),
                            preferred_element_type=jnp.float32)
    col = i * BN + jax.lax.broadcasted_iota(jnp.int32, d.shape, 1)
    a = jnp.exp(d - m_ref[...]) / s_ref[...]
    a = jnp.where(col < n_valid, a, 0.0)
    a_out[...] = a
    y_sc[...] += jax.lax.dot_general(a, xyz_ref[...],
                                     (((1,), (0,)), ((), ())),
                                     preferred_element_type=jnp.float32)

    @pl.when(i == nb - 1)
    def _fin():
        y_out[...] = y_sc[...]


def _readout(hs_rec, hs_lig_k, xyz_rec):
    n = hs_rec.shape[0]
    k = hs_lig_k.shape[0]
    n_pad = ((n + BN - 1) // BN) * BN
    if n_pad != n:
        hs_rec = jnp.pad(hs_rec, ((0, n_pad - n), (0, 0)))
        xyz_rec = jnp.pad(xyz_rec, ((0, n_pad - n), (0, 0)))
    nb = n_pad // BN
    m, s = pl.pallas_call(
        functools.partial(_stats_body, n),
        grid=(nb,),
        in_specs=[
            pl.BlockSpec((k, D), lambda i: (0, 0)),
            pl.BlockSpec((BN, D), lambda i: (i, 0)),
        ],
        out_specs=[
            pl.BlockSpec((k, 1), lambda i: (0, 0)),
            pl.BlockSpec((k, 1), lambda i: (0, 0)),
        ],
        out_shape=[
            jax.ShapeDtypeStruct((k, 1), jnp.float32),
            jax.ShapeDtypeStruct((k, 1), jnp.float32),
        ],
        scratch_shapes=[
            pltpu.VMEM((k, 1), jnp.float32),
            pltpu.VMEM((k, 1), jnp.float32),
        ],
    )(hs_lig_k, hs_rec)
    a, y = pl.pallas_call(
        functools.partial(_readout_body, n),
        grid=(nb,),
        in_specs=[
            pl.BlockSpec((k, D), lambda i: (0, 0)),
            pl.BlockSpec((k, 1), lambda i: (0, 0)),
            pl.BlockSpec((k, 1), lambda i: (0, 0)),
            pl.BlockSpec((BN, D), lambda i: (i, 0)),
            pl.BlockSpec((BN, 3), lambda i: (i, 0)),
        ],
        out_specs=[
            pl.BlockSpec((k, BN), lambda i: (0, i)),
            pl.BlockSpec((k, 3), lambda i: (0, 0)),
        ],
        out_shape=[
            jax.ShapeDtypeStruct((k, n_pad), jnp.float32),
            jax.ShapeDtypeStruct((k, 3), jnp.float32),
        ],
        scratch_shapes=[
            pltpu.VMEM((k, 3), jnp.float32),
        ],
    )(hs_lig_k, m, s, hs_rec, xyz_rec)
    return y, a[:, :n]


def _head_mask():
    return (jnp.arange(D)[:, None] // (D // H)
            == jnp.arange(H)[None, :]).astype(jnp.float32)


def _attn_layer_sc(h, src, dst, eattr, p, n, zeros_n16, zeros_acc):
    d_e = eattr.shape[1]
    hmask = _head_mask()
    sw_used = H * d_e
    qw = D + ((sw_used + 15) // 16) * 16
    wek = p['Wek']
    if d_e > 1:
        wek = wek.astype(jnp.bfloat16).astype(jnp.float32)
    ws = jnp.concatenate(
        [wek[j][:, None] * hmask for j in range(d_e)], axis=1)
    ws = jnp.pad(ws, ((0, 0), (0, (qw - D) - sw_used)))

    qs, kn, vn = _proj(h, p['Wq'], p['Wk'], p['Wv'], ws, qw)
    ksrc, qsdst = _sc_gather_kq(kn, qs, src, dst)
    el16 = _edge_logits(ksrc, qsdst, eattr, hmask)
    dpart = _sc_scatter_denom(el16, dst, n, zeros_n16)
    denom = _add2(dpart[0], dpart[1])
    ddst, vsrc = _sc_gather_dv(denom, vn, src, dst)
    msg = _msg(el16, ddst, vsrc, eattr, p['Wev'], hmask.T)
    agg = _sc_scatter_msg(msg, dst, n, zeros_acc)
    return _epi(h, agg, p['Ws'], p['Wo'])


def _encode_sc(attr, eattr, eidx, p, n):
    src, dst = eidx[0], eidx[1]
    zeros_n16 = jnp.zeros((n, 16), jnp.float32)
    zeros_acc = jnp.zeros((n // NC + 8, D), jnp.float32)
    zb = jnp.zeros((1, D), jnp.float32)
    h = _lin_relu(attr, p['Win'], zb)
    for lp in p['layers']:
        h = _attn_layer_sc(h, src, dst, eattr, lp, n, zeros_n16, zeros_acc)
    return _mm(h, p['Wout'])


def kernel(rec_attr, rec_edge_attr, rec_x, lig_attr, lig_edge_attr, labelidx,
           params, rec_edge_index, lig_edge_index):
    hs_rec = _encode_sc(rec_attr, rec_edge_attr, rec_edge_index,
                        params['rec'], rec_attr.shape[0])
    hs_lig = _encode_sc(lig_attr, lig_edge_attr, lig_edge_index,
                        params['lig'], lig_attr.shape[0])
    xyz_rec = rec_x.reshape(-1, 3)
    hs_lig_k = _label_contract(labelidx, hs_lig, params['phi_W'],
                               params['phi_b'])
    return _readout(hs_rec, hs_lig_k, xyz_rec)


# speedup vs baseline: 3.0056x; 1.0025x over previous
import functools

import jax
import jax.numpy as jnp
import numpy as np
from jax import lax
from jax.experimental import pallas as pl
from jax.experimental.pallas import tpu as pltpu
from jax.experimental.pallas import tpu_sc as plsc

D = 32
H = 4
NC = 2
NS = 16
NW = NC * NS
BE = 5000
BN = 4096


def _pick_chunk(per, align):
    for c in (5000, 4000, 2000, 1600, 1000, 800, 512, 400, 256, 200, 160, 128, 100, 80, 64, 32, 16):
        if c % align == 0 and per % c == 0:
            return c
    raise ValueError(f"no chunk for {per} align {align}")


def _mesh():
    return plsc.VectorSubcoreMesh(core_axis_name="c", subcore_axis_name="s",
                                  num_cores=NC, num_subcores=NS)


_SC_PARAMS = pltpu.CompilerParams(use_tc_tiling_on_sc=False)


def _sc_gather_kq(kn, qs, src, dst):
    e = src.shape[0]
    qw = qs.shape[1]
    per_w = e // NW
    c_sz = 400 if per_w % 400 == 0 else 200
    rounds = per_w // c_sz

    @functools.partial(
        pl.kernel,
        out_type=[jax.ShapeDtypeStruct((e, D), jnp.float32),
                  jax.ShapeDtypeStruct((e, qw), jnp.float32)],
        mesh=_mesh(),
        compiler_params=_SC_PARAMS,
        scratch_types=[pltpu.VMEM((2, c_sz), jnp.int32),
                       pltpu.VMEM((2, c_sz), jnp.int32),
                       pltpu.VMEM((2, c_sz, D), jnp.float32),
                       pltpu.VMEM((2, c_sz, qw), jnp.float32),
                       pltpu.SemaphoreType.DMA,
                       pltpu.SemaphoreType.DMA,
                       pltpu.SemaphoreType.DMA,
                       pltpu.SemaphoreType.DMA],
    )
    def k(kn_h, qs_h, src_h, dst_h, ko, qo, si, di, kb, qb, s1, s2, s3, s4):
        w = lax.axis_index("s") * NC + lax.axis_index("c")
        base0 = w * per_w

        def start(r, p):
            base = base0 + r * c_sz
            pltpu.sync_copy(src_h.at[pl.ds(base, c_sz)], si.at[p])
            pltpu.sync_copy(dst_h.at[pl.ds(base, c_sz)], di.at[p])
            return (pltpu.async_copy(kn_h.at[si.at[p]], kb.at[p],
                                     s1 if p == 0 else s3),
                    pltpu.async_copy(qs_h.at[di.at[p]], qb.at[p],
                                     s2 if p == 0 else s4))

        def finish(r, p, cps):
            base = base0 + r * c_sz
            cps[0].wait()
            cps[1].wait()
            pltpu.sync_copy(kb.at[p], ko.at[pl.ds(base, c_sz)])
            pltpu.sync_copy(qb.at[p], qo.at[pl.ds(base, c_sz)])

        def body(i, carry):
            r = i * 2
            ca = start(r, 0)
            cb = start(r + 1, 1)
            finish(r, 0, ca)
            finish(r + 1, 1, cb)
            return carry

        lax.fori_loop(0, rounds // 2, body, 0)
        if rounds % 2:
            finish(rounds - 1, 0, start(rounds - 1, 0))

    return k(kn, qs, src, dst)


def _sc_gather_dv(den, vn, src, dst):
    e = src.shape[0]
    per_w = e // NW
    c_sz = 1000
    assert per_w % c_sz == 0
    rounds = per_w // c_sz

    @functools.partial(
        pl.kernel,
        out_type=[jax.ShapeDtypeStruct((e, 16), jnp.float32),
                  jax.ShapeDtypeStruct((e, D), jnp.float32)],
        mesh=_mesh(),
        compiler_params=_SC_PARAMS,
        scratch_types=[pltpu.VMEM((c_sz,), jnp.int32),
                       pltpu.VMEM((c_sz,), jnp.int32),
                       pltpu.VMEM((c_sz, 16), jnp.float32),
                       pltpu.VMEM((c_sz, D), jnp.float32),
                       pltpu.SemaphoreType.DMA,
                       pltpu.SemaphoreType.DMA],
    )
    def k(den_h, vn_h, src_h, dst_h, do, vo, si, di, db, vb, s1, s2):
        w = lax.axis_index("s") * NC + lax.axis_index("c")
        base0 = w * per_w

        def body(r, carry):
            base = base0 + r * c_sz
            pltpu.sync_copy(src_h.at[pl.ds(base, c_sz)], si)
            pltpu.sync_copy(dst_h.at[pl.ds(base, c_sz)], di)
            c1 = pltpu.async_copy(den_h.at[di], db, s1)
            c2 = pltpu.async_copy(vn_h.at[si], vb, s2)
            c1.wait()
            c2.wait()
            pltpu.sync_copy(db, do.at[pl.ds(base, c_sz)])
            pltpu.sync_copy(vb, vo.at[pl.ds(base, c_sz)])
            return carry

        lax.fori_loop(0, rounds, body, 0)

    return k(den, vn, src, dst)


def _sc_scatter_denom(el16, dst, n, zeros_n16):
    e = dst.shape[0]
    per_w = e // NW
    c_sz = 1000
    assert per_w % c_sz == 0
    rounds = per_w // c_sz

    @functools.partial(
        pl.kernel,
        out_type=jax.ShapeDtypeStruct((NC, n, 16), jnp.float32),
        mesh=_mesh(),
        compiler_params=_SC_PARAMS,
        scratch_types=[pltpu.VMEM((c_sz,), jnp.int32),
                       pltpu.VMEM((c_sz, 16), jnp.float32),
                       pltpu.VMEM_SHARED((n, 16), jnp.float32)],
    )
    def k(el_h, dst_h, z_h, out, iv, rb, acc):
        c = lax.axis_index("c")
        s = lax.axis_index("s")
        w = s * NC + c

        @pl.when(s == 0)
        def _zero():
            pltpu.sync_copy(z_h, acc)

        plsc.subcore_barrier()

        def body(r, carry):
            base = w * per_w + r * c_sz
            pltpu.sync_copy(dst_h.at[pl.ds(base, c_sz)], iv)
            pltpu.sync_copy(el_h.at[pl.ds(base, c_sz)], rb)
            pltpu.sync_copy(rb, acc.at[iv], add=True)
            return carry

        lax.fori_loop(0, rounds, body, 0)
        plsc.subcore_barrier()

        @pl.when(s == 0)
        def _dump():
            pltpu.sync_copy(acc, out.at[c])

    return k(el16, dst, zeros_n16)


def _sc_scatter_msg(msg, dst, n, zeros_acc):
    e = dst.shape[0]
    n2 = n // NC
    npad = n2 + 8
    per_t = e // NS
    c_sz = 400
    assert per_t % c_sz == 0
    rounds = per_t // c_sz

    @functools.partial(
        pl.kernel,
        out_type=jax.ShapeDtypeStruct((n, D), jnp.float32),
        mesh=_mesh(),
        compiler_params=_SC_PARAMS,
        scratch_types=[pltpu.VMEM((2, c_sz), jnp.int32),
                       pltpu.VMEM((2, c_sz, D), jnp.float32),
                       pltpu.VMEM_SHARED((npad, D), jnp.float32),
                       pltpu.SemaphoreType.DMA,
                       pltpu.SemaphoreType.DMA,
                       pltpu.SemaphoreType.DMA,
                       pltpu.SemaphoreType.DMA],
    )
    def k(msg_h, dst_h, z_h, out, iv, rb, acc, s1, s2, s3, s4):
        c = lax.axis_index("c")
        s = lax.axis_index("s")
        lo = c * n2

        @pl.when(s == 0)
        def _zero():
            pltpu.sync_copy(z_h, acc)

        plsc.subcore_barrier()

        def start(r, p):
            base = s * per_t + r * c_sz
            return (pltpu.async_copy(dst_h.at[pl.ds(base, c_sz)], iv.at[p],
                                     s1 if p == 0 else s3),
                    pltpu.async_copy(msg_h.at[pl.ds(base, c_sz)], rb.at[p],
                                     s2 if p == 0 else s4))

        def finish(p, cps):
            cps[0].wait()
            cps[1].wait()
            for g in range(c_sz // 16):
                v = iv[p, pl.ds(g * 16, 16)]
                loc = v - lo
                ok = (loc >= 0) & (loc < n2)
                iv[p, pl.ds(g * 16, 16)] = jnp.where(ok, loc, n2)
            pltpu.sync_copy(rb.at[p], acc.at[iv.at[p]], add=True)

        def body(i, carry):
            r = i * 2
            ca = start(r, 0)
            cb = start(r + 1, 1)
            finish(0, ca)
            finish(1, cb)
            return carry

        lax.fori_loop(0, rounds // 2, body, 0)
        if rounds % 2:
            finish(0, start(rounds - 1, 0))
        plsc.subcore_barrier()

        @pl.when(s == 0)
        def _dump():
            pltpu.sync_copy(acc.at[pl.ds(0, n2)], out.at[pl.ds(lo, n2)])

    return k(msg, dst, zeros_acc)


def _grid1(n_rows, blk):
    assert n_rows % blk == 0, (n_rows, blk)
    return n_rows // blk


def _row_spec(blk, w):
    return pl.BlockSpec((blk, w), lambda i: (i, 0))


def _full_spec(a, b):
    return pl.BlockSpec((a, b), lambda i: (0, 0))


def _lin_relu_body(a_ref, w_ref, b_ref, o_ref):
    o_ref[...] = jax.nn.relu(
        jax.lax.dot_general(a_ref[...], w_ref[...], (((1,), (0,)), ((), ())),
                            preferred_element_type=jnp.float32) + b_ref[...])


def _lin_relu(a, w, b):
    n, din = a.shape
    g = _grid1(n, BE)
    return pl.pallas_call(
        _lin_relu_body,
        grid=(g,),
        in_specs=[_row_spec(BE, din), _full_spec(din, D), _full_spec(1, D)],
        out_specs=_row_spec(BE, D),
        out_shape=jax.ShapeDtypeStruct((n, D), jnp.float32),
    )(a, w, b.reshape(1, D))


def _mm_body(a_ref, w_ref, o_ref):
    o_ref[...] = jax.lax.dot_general(
        a_ref[...], w_ref[...], (((1,), (0,)), ((), ())),
        preferred_element_type=jnp.float32)


def _mm(a, w):
    n = a.shape[0]
    g = _grid1(n, BE)
    return pl.pallas_call(
        _mm_body,
        grid=(g,),
        in_specs=[_row_spec(BE, D), _full_spec(D, w.shape[1])],
        out_specs=_row_spec(BE, w.shape[1]),
        out_shape=jax.ShapeDtypeStruct((n, w.shape[1]), jnp.float32),
    )(a, w)


def _proj_body(h_ref, wq, wk, wv, wsp, qs_o, k_o, v_o):
    hh = h_ref[...]
    dn = (((1,), (0,)), ((), ()))
    q = jax.lax.dot_general(hh, wq[...], dn, preferred_element_type=jnp.float32)
    k_o[...] = jax.lax.dot_general(hh, wk[...], dn,
                                   preferred_element_type=jnp.float32)
    v_o[...] = jax.lax.dot_general(hh, wv[...], dn,
                                   preferred_element_type=jnp.float32)
    s = jax.lax.dot_general(q, wsp[...], dn,
                            preferred_element_type=jnp.float32,
                            precision=lax.Precision.HIGHEST)
    qs_o[...] = jnp.concatenate([q, s], axis=1)


def _proj(h, wq, wk, wv, wsp, qw):
    n = h.shape[0]
    g = _grid1(n, BE)
    sw = qw - D
    return pl.pallas_call(
        _proj_body,
        grid=(g,),
        in_specs=[_row_spec(BE, D), _full_spec(D, D), _full_spec(D, D),
                  _full_spec(D, D), _full_spec(D, sw)],
        out_specs=[_row_spec(BE, qw), _row_spec(BE, D), _row_spec(BE, D)],
        out_shape=[jax.ShapeDtypeStruct((n, qw), jnp.float32),
                   jax.ShapeDtypeStruct((n, D), jnp.float32),
                   jax.ShapeDtypeStruct((n, D), jnp.float32)],
    )(h, wq, wk, wv, wsp)


def _edge_logits_body(d_e, ks_ref, qs_ref, ea_ref, hm_ref, el_o):
    q = qs_ref[:, :D]
    p = q * ks_ref[...]
    l = jax.lax.dot_general(p, hm_ref[...], (((1,), (0,)), ((), ())),
                            preferred_element_type=jnp.float32,
                            precision=lax.Precision.HIGHEST)
    ea = ea_ref[...]
    if d_e > 1:
        ea = ea.astype(jnp.bfloat16).astype(jnp.float32)
    for j in range(d_e):
        l = l + ea[:, j:j + 1] * qs_ref[:, D + H * j:D + H * j + H]
    l = l * np.float32(1.0 / np.sqrt(D // H))
    el = jnp.exp(l)
    el_o[...] = jnp.concatenate(
        [el, jnp.zeros((el.shape[0], 12), jnp.float32)], axis=1)


def _edge_logits(ksrc, qsdst, eattr, hmask):
    e, d_e = eattr.shape
    qw = qsdst.shape[1]
    g = _grid1(e, BE)
    return pl.pallas_call(
        functools.partial(_edge_logits_body, d_e),
        grid=(g,),
        in_specs=[_row_spec(BE, D), _row_spec(BE, qw),
                  _row_spec(BE, d_e), _full_spec(D, H)],
        out_specs=_row_spec(BE, 16),
        out_shape=jax.ShapeDtypeStruct((e, 16), jnp.float32),
    )(ksrc, qsdst, eattr, hmask)


def _add2_body(a_ref, b_ref, o_ref):
    o_ref[...] = a_ref[...] + b_ref[...]


def _add2(a, b):
    n = a.shape[0]
    g = _grid1(n, BE)
    return pl.pallas_call(
        _add2_body,
        grid=(g,),
        in_specs=[_row_spec(BE, a.shape[1]), _row_spec(BE, a.shape[1])],
        out_specs=_row_spec(BE, a.shape[1]),
        out_shape=jax.ShapeDtypeStruct(a.shape, jnp.float32),
    )(a, b)


def _msg_body(d_e, el_ref, dd_ref, vs_ref, ea_ref, wev_ref, r_ref, o_ref):
    w = el_ref[:, :H] / (dd_ref[:, :H] + 1e-9)
    ea = ea_ref[...]
    wev = wev_ref[...]
    if d_e > 1:
        ea = ea.astype(jnp.bfloat16).astype(jnp.float32)
        wev = wev.astype(jnp.bfloat16).astype(jnp.float32)
    ve = vs_ref[...]
    for j in range(d_e):
        ve = ve + ea[:, j:j + 1] * wev[j:j + 1, :]
    o_ref[...] = jax.lax.dot_general(
        w, r_ref[...], (((1,), (0,)), ((), ())),
        preferred_element_type=jnp.float32,
        precision=lax.Precision.HIGHEST) * ve


def _msg(el16, ddst, vsrc, eattr, wev, rexp):
    e, d_e = eattr.shape
    g = _grid1(e, BE)
    return pl.pallas_call(
        functools.partial(_msg_body, d_e),
        grid=(g,),
        in_specs=[_row_spec(BE, 16), _row_spec(BE, 16), _row_spec(BE, D),
                  _row_spec(BE, d_e), _full_spec(d_e, D), _full_spec(H, D)],
        out_specs=_row_spec(BE, D),
        out_shape=jax.ShapeDtypeStruct((e, D), jnp.float32),
    )(el16, ddst, vsrc, eattr, wev, rexp)


def _epi_body(h_ref, agg_ref, ws, wo, o_ref):
    dn = (((1,), (0,)), ((), ()))
    o_ref[...] = jax.nn.relu(
        jax.lax.dot_general(h_ref[...], ws[...], dn,
                            preferred_element_type=jnp.float32)
        + jax.lax.dot_general(agg_ref[...], wo[...], dn,
                              preferred_element_type=jnp.float32))


def _epi(h, agg, ws, wo):
    n = h.shape[0]
    g = _grid1(n, BE)
    return pl.pallas_call(
        _epi_body,
        grid=(g,),
        in_specs=[_row_spec(BE, D), _row_spec(BE, D), _full_spec(D, D),
                  _full_spec(D, D)],
        out_specs=_row_spec(BE, D),
        out_shape=jax.ShapeDtypeStruct((n, D), jnp.float32),
    )(h, agg, ws, wo)


def _lab_body(li_ref, hl_ref, pw, pb, o_ref):
    dn = (((1,), (0,)), ((), ()))
    hl = jax.nn.relu(
        jax.lax.dot_general(hl_ref[...], pw[...], dn,
                            preferred_element_type=jnp.float32) + pb[...])
    o_ref[...] = jax.lax.dot_general(li_ref[...], hl, dn,
                                     preferred_element_type=jnp.float32)


def _label_contract(labelidx, hs_lig, phi_w, phi_b):
    k, n = labelidx.shape
    return pl.pallas_call(
        _lab_body,
        grid=(1,),
        in_specs=[_full_spec(k, n), _full_spec(n, D),
                  _full_spec(D, D), _full_spec(1, D)],
        out_specs=pl.BlockSpec((k, D), lambda i: (0, 0)),
        out_shape=jax.ShapeDtypeStruct((k, D), jnp.float32),
    )(labelidx, hs_lig, phi_w, phi_b.reshape(1, D))


def _stats_body(n_valid, lig_ref, rec_ref, m_out, s_out, m_sc, s_sc):
    i = pl.program_id(0)
    nb = pl.num_programs(0)

    @pl.when(i == 0)
    def _init():
        m_sc[...] = jnp.full_like(m_sc, -jnp.inf)
        s_sc[...] = jnp.zeros_like(s_sc)

    d = jax.lax.dot_general(lig_ref[...], rec_ref[...],
                            (((1,), (1,)), ((), ())),
                            preferred_element_type=jnp.float32)
    col = i * BN + jax.lax.broadcasted_iota(jnp.int32, d.shape, 1)
    d = jnp.where(col < n_valid, d, -jnp.inf)
    bm = jnp.max(d, axis=1, keepdims=True)
    m_old = m_sc[...]
    m_new = jnp.maximum(m_old, bm)
    s_sc[...] = s_sc[...] * jnp.exp(m_old - m_new) + jnp.sum(
        jnp.exp(d - m_new), axis=1, keepdims=True)
    m_sc[...] = m_new

    @pl.when(i == nb - 1)
    def _fin():
        m_out[...] = m_sc[...]
        s_out[...] = s_sc[...]


def _readout_body(n_valid, lig_ref, m_ref, s_ref, rec_ref, xyz_ref, a_out,
                  y_out, y_sc):
    i = pl.program_id(0)
    nb = pl.num_programs(0)

    @pl.when(i == 0)
    def _init():
        y_sc[...] = jnp.zeros_like(y_sc)

    d = jax.lax.dot_general(lig_ref[...], rec_ref[...],
                            (((1,), (1,)), ((), ())),
                            preferred_element_type=jnp.float32)
    col = i * BN + jax.lax.broadcasted_iota(jnp.int32, d.shape, 1)
    a = jnp.exp(d - m_ref[...]) / s_ref[...]
    a = jnp.where(col < n_valid, a, 0.0)
    a_out[...] = a
    y_sc[...] += jax.lax.dot_general(a, xyz_ref[...],
                                     (((1,), (0,)), ((), ())),
                                     preferred_element_type=jnp.float32)

    @pl.when(i == nb - 1)
    def _fin():
        y_out[...] = y_sc[...]


def _readout(hs_rec, hs_lig_k, xyz_rec):
    n = hs_rec.shape[0]
    k = hs_lig_k.shape[0]
    n_pad = ((n + BN - 1) // BN) * BN
    if n_pad != n:
        hs_rec = jnp.pad(hs_rec, ((0, n_pad - n), (0, 0)))
        xyz_rec = jnp.pad(xyz_rec, ((0, n_pad - n), (0, 0)))
    nb = n_pad // BN
    m, s = pl.pallas_call(
        functools.partial(_stats_body, n),
        grid=(nb,),
        in_specs=[
            pl.BlockSpec((k, D), lambda i: (0, 0)),
            pl.BlockSpec((BN, D), lambda i: (i, 0)),
        ],
        out_specs=[
            pl.BlockSpec((k, 1), lambda i: (0, 0)),
            pl.BlockSpec((k, 1), lambda i: (0, 0)),
        ],
        out_shape=[
            jax.ShapeDtypeStruct((k, 1), jnp.float32),
            jax.ShapeDtypeStruct((k, 1), jnp.float32),
        ],
        scratch_shapes=[
            pltpu.VMEM((k, 1), jnp.float32),
            pltpu.VMEM((k, 1), jnp.float32),
        ],
    )(hs_lig_k, hs_rec)
    a, y = pl.pallas_call(
        functools.partial(_readout_body, n),
        grid=(nb,),
        in_specs=[
            pl.BlockSpec((k, D), lambda i: (0, 0)),
            pl.BlockSpec((k, 1), lambda i: (0, 0)),
            pl.BlockSpec((k, 1), lambda i: (0, 0)),
            pl.BlockSpec((BN, D), lambda i: (i, 0)),
            pl.BlockSpec((BN, 3), lambda i: (i, 0)),
        ],
        out_specs=[
            pl.BlockSpec((k, BN), lambda i: (0, i)),
            pl.BlockSpec((k, 3), lambda i: (0, 0)),
        ],
        out_shape=[
            jax.ShapeDtypeStruct((k, n_pad), jnp.float32),
            jax.ShapeDtypeStruct((k, 3), jnp.float32),
        ],
        scratch_shapes=[
            pltpu.VMEM((k, 3), jnp.float32),
        ],
    )(hs_lig_k, m, s, hs_rec, xyz_rec)
    return y, a[:, :n]


def _head_mask():
    return (jnp.arange(D)[:, None] // (D // H)
            == jnp.arange(H)[None, :]).astype(jnp.float32)


def _attn_layer_sc(h, src, dst, eattr, p, n, zeros_n16, zeros_acc):
    d_e = eattr.shape[1]
    hmask = _head_mask()
    sw_used = H * d_e
    qw = D + ((sw_used + 15) // 16) * 16
    wek = p['Wek']
    if d_e > 1:
        wek = wek.astype(jnp.bfloat16).astype(jnp.float32)
    ws = jnp.concatenate(
        [wek[j][:, None] * hmask for j in range(d_e)], axis=1)
    ws = jnp.pad(ws, ((0, 0), (0, (qw - D) - sw_used)))

    qs, kn, vn = _proj(h, p['Wq'], p['Wk'], p['Wv'], ws, qw)
    ksrc, qsdst = _sc_gather_kq(kn, qs, src, dst)
    el16 = _edge_logits(ksrc, qsdst, eattr, hmask)
    dpart = _sc_scatter_denom(el16, dst, n, zeros_n16)
    denom = _add2(dpart[0], dpart[1])
    ddst, vsrc = _sc_gather_dv(denom, vn, src, dst)
    msg = _msg(el16, ddst, vsrc, eattr, p['Wev'], hmask.T)
    agg = _sc_scatter_msg(msg, dst, n, zeros_acc)
    return _epi(h, agg, p['Ws'], p['Wo'])


def _encode_sc(attr, eattr, eidx, p, n):
    src, dst = eidx[0], eidx[1]
    zeros_n16 = jnp.zeros((n, 16), jnp.float32)
    zeros_acc = jnp.zeros((n // NC + 8, D), jnp.float32)
    zb = jnp.zeros((1, D), jnp.float32)
    h = _lin_relu(attr, p['Win'], zb)
    for lp in p['layers']:
        h = _attn_layer_sc(h, src, dst, eattr, lp, n, zeros_n16, zeros_acc)
    return _mm(h, p['Wout'])


def kernel(rec_attr, rec_edge_attr, rec_x, lig_attr, lig_edge_attr, labelidx,
           params, rec_edge_index, lig_edge_index):
    hs_rec = _encode_sc(rec_attr, rec_edge_attr, rec_edge_index,
                        params['rec'], rec_attr.shape[0])
    hs_lig = _encode_sc(lig_attr, lig_edge_attr, lig_edge_index,
                        params['lig'], lig_attr.shape[0])
    xyz_rec = rec_x.reshape(-1, 3)
    hs_lig_k = _label_contract(labelidx, hs_lig, params['phi_W'],
                               params['phi_b'])
    return _readout(hs_rec, hs_lig_k, xyz_rec)
